# baseline (speedup 1.0000x reference)
import jax
import jax.numpy as jnp
from jax.experimental import pallas as pl

B, N, D = 4, 2048, 512
PD = D // 4
TOP_K = 32
THRESH = 0.7
SCALE = float(PD) ** 0.5
INT_MIN = -(2 ** 31)

RB = 1024


def _proj_kernel(a_ref, st_ref, wq_ref, bq_ref, wk_ref, bk_ref, wc_ref, bc_ref,
                 q_ref, k_ref, pn_ref, stb_ref):
    a = a_ref[0]
    q = jnp.dot(a, wq_ref[...], preferred_element_type=jnp.float32) + bq_ref[...]
    k = jnp.dot(a, wk_ref[...], preferred_element_type=jnp.float32) + bk_ref[...]
    c = jnp.dot(a, wc_ref[...], preferred_element_type=jnp.float32) + bc_ref[...]
    norm = jnp.sqrt(jnp.sum(c * c, axis=-1, keepdims=True))
    pn = c / jnp.maximum(norm, 1e-12)
    q_ref[0] = q
    k_ref[0] = k
    pn_ref[0] = pn.astype(jnp.bfloat16)
    stb_ref[0] = st_ref[0].astype(jnp.bfloat16)


def _messages_kernel(q_ref, k_ref, stb_ref, m_ref, mb_ref):
    q = q_ref[0]
    k = k_ref[0]
    s = jax.lax.dot_general(q, k, (((1,), (1,)), ((), ())),
                            preferred_element_type=jnp.float32) * (1.0 / SCALE)

    rowmax = jnp.max(s, axis=1, keepdims=True)

    u = jax.lax.bitcast_convert_type(s, jnp.int32)
    key = jnp.where(u >= 0, u, jnp.int32(INT_MIN) - u)

    khi = jax.lax.shift_right_arithmetic(key, 16).astype(jnp.int16)

    def _sum16(mask16):
        m = mask16
        w = N
        while w > 1:
            w //= 2
            m = m[:, :w] + m[:, w:2 * w]
        return m.astype(jnp.int32)

    cnt0 = _sum16((khi >= 0).astype(jnp.int16))
    base0 = jnp.where(cnt0 >= TOP_K, jnp.int32(0), jnp.int32(-32768))

    def body_hi(i, base):
        cand = base + (jnp.int32(1) << (14 - i))
        cnt = _sum16((khi >= cand.astype(jnp.int16)).astype(jnp.int16))
        return jnp.where(cnt >= TOP_K, cand, base)

    hi = jax.lax.fori_loop(0, 15, body_hi, base0)
    hi16 = hi.astype(jnp.int16)

    cgt = _sum16((khi > hi16).astype(jnp.int16))
    rem = jnp.int32(TOP_K) - cgt
    lo16 = (jnp.bitwise_and(key, 0xFFFF) - 32768).astype(jnp.int16)
    kl = jnp.where(khi == hi16, lo16, jnp.int16(-32768))

    def body_lo(i, base):
        cand = base + (jnp.int32(1) << (15 - i))
        cnt = _sum16((kl >= cand.astype(jnp.int16)).astype(jnp.int16))
        return jnp.where(cnt >= rem, cand, base)

    lo = jax.lax.fori_loop(0, 16, body_lo, jnp.full_like(hi, -32768))

    kth_key = (hi << 16) + (lo + 32768)

    p = jnp.where(key >= kth_key, jnp.exp(s - rowmax), 0.0)
    denom = jnp.sum(p, axis=1, keepdims=True)
    msg = jax.lax.dot_general(
        p.astype(jnp.bfloat16), stb_ref[0], (((1,), (0,)), ((), ())),
        preferred_element_type=jnp.float32) / denom
    m_ref[0] = msg
    mb_ref[0] = msg.astype(jnp.bfloat16)


def _coalition_kernel(pnb_ref, pnf_ref, msgb_ref, msgblk_ref, o_ref):
    sim = jax.lax.dot_general(pnb_ref[0], pnf_ref[0], (((1,), (1,)), ((), ())),
                              preferred_element_type=jnp.float32)
    hit = sim > THRESH
    cnt = jnp.sum(hit.astype(jnp.float32), axis=1, keepdims=True)
    comb = jax.lax.dot_general(
        hit.astype(jnp.bfloat16), msgb_ref[0], (((1,), (0,)), ((), ())),
        preferred_element_type=jnp.float32) / (cnt + 1e-8)
    o_ref[0] = 0.8 * msgblk_ref[0] + 0.2 * comb


@jax.jit
def kernel(agent_states, agent_actions, Wq, bq, Wk, bk, Wc, bc):
    nb = N // RB
    q, k, pn, stb = pl.pallas_call(
        _proj_kernel,
        grid=(B,),
        in_specs=[
            pl.BlockSpec((1, N, D), lambda b: (b, 0, 0)),
            pl.BlockSpec((1, N, D), lambda b: (b, 0, 0)),
            pl.BlockSpec((D, PD), lambda b: (0, 0)),
            pl.BlockSpec((PD,), lambda b: (0,)),
            pl.BlockSpec((D, PD), lambda b: (0, 0)),
            pl.BlockSpec((PD,), lambda b: (0,)),
            pl.BlockSpec((D, PD), lambda b: (0, 0)),
            pl.BlockSpec((PD,), lambda b: (0,)),
        ],
        out_specs=[
            pl.BlockSpec((1, N, PD), lambda b: (b, 0, 0)),
            pl.BlockSpec((1, N, PD), lambda b: (b, 0, 0)),
            pl.BlockSpec((1, N, PD), lambda b: (b, 0, 0)),
            pl.BlockSpec((1, N, D), lambda b: (b, 0, 0)),
        ],
        out_shape=[
            jax.ShapeDtypeStruct((B, N, PD), jnp.float32),
            jax.ShapeDtypeStruct((B, N, PD), jnp.float32),
            jax.ShapeDtypeStruct((B, N, PD), jnp.bfloat16),
            jax.ShapeDtypeStruct((B, N, D), jnp.bfloat16),
        ],
    )(agent_actions, agent_states, Wq, bq, Wk, bk, Wc, bc)

    messages, messages_b = pl.pallas_call(
        _messages_kernel,
        grid=(B, nb),
        in_specs=[
            pl.BlockSpec((1, RB, PD), lambda b, i: (b, i, 0)),
            pl.BlockSpec((1, N, PD), lambda b, i: (b, 0, 0)),
            pl.BlockSpec((1, N, D), lambda b, i: (b, 0, 0)),
        ],
        out_specs=[
            pl.BlockSpec((1, RB, D), lambda b, i: (b, i, 0)),
            pl.BlockSpec((1, RB, D), lambda b, i: (b, i, 0)),
        ],
        out_shape=[
            jax.ShapeDtypeStruct((B, N, D), jnp.float32),
            jax.ShapeDtypeStruct((B, N, D), jnp.bfloat16),
        ],
    )(q, k, stb)

    out = pl.pallas_call(
        _coalition_kernel,
        grid=(B, nb),
        in_specs=[
            pl.BlockSpec((1, RB, PD), lambda b, i: (b, i, 0)),
            pl.BlockSpec((1, N, PD), lambda b, i: (b, 0, 0)),
            pl.BlockSpec((1, N, D), lambda b, i: (b, 0, 0)),
            pl.BlockSpec((1, RB, D), lambda b, i: (b, i, 0)),
        ],
        out_specs=pl.BlockSpec((1, RB, D), lambda b, i: (b, i, 0)),
        out_shape=jax.ShapeDtypeStruct((B, N, D), jnp.float32),
    )(pn, pn, messages_b, messages)

    return out


# speedup vs baseline: 1.4325x; 1.4325x over previous
import jax
import jax.numpy as jnp
from jax.experimental import pallas as pl

B, N, D = 4, 2048, 512
PD = D // 4
TOP_K = 32
THRESH = 0.7
SCALE = float(PD) ** 0.5
INT_MIN = -(2 ** 31)

RB = 1024


def _proj_kernel(a_ref, st_ref, wq_ref, bq_ref, wk_ref, bk_ref, wc_ref, bc_ref,
                 q_ref, k_ref, pn_ref, stb_ref):
    a = a_ref[0]
    q = jnp.dot(a, wq_ref[...], preferred_element_type=jnp.float32) + bq_ref[...]
    k = jnp.dot(a, wk_ref[...], preferred_element_type=jnp.float32) + bk_ref[...]
    c = jnp.dot(a, wc_ref[...], preferred_element_type=jnp.float32) + bc_ref[...]
    norm = jnp.sqrt(jnp.sum(c * c, axis=-1, keepdims=True))
    pn = c / jnp.maximum(norm, 1e-12)
    q_ref[0] = q
    k_ref[0] = k
    pn_ref[0] = pn.astype(jnp.bfloat16)
    stb_ref[0] = st_ref[0].astype(jnp.bfloat16)


def _messages_kernel(q_ref, k_ref, stb_ref, m_ref, mb_ref):
    q = q_ref[0]
    k = k_ref[0]
    s = jax.lax.dot_general(q, k, (((1,), (1,)), ((), ())),
                            preferred_element_type=jnp.float32) * (1.0 / SCALE)

    rowmax = jnp.max(s, axis=1, keepdims=True)

    u = jax.lax.bitcast_convert_type(s, jnp.int32)
    key = jnp.where(u >= 0, u, jnp.int32(INT_MIN) - u)

    khi = jax.lax.shift_right_arithmetic(key, 16).astype(jnp.int16)

    def _sum16(mask16):
        m = mask16
        w = N
        while w > 128:
            w //= 2
            m = m[:, :w] + m[:, w:2 * w]
        return jnp.sum(m.astype(jnp.float32), axis=1, keepdims=True)

    cnt0 = _sum16((khi >= 0).astype(jnp.int16))
    base0 = jnp.where(cnt0 >= TOP_K, jnp.int32(0), jnp.int32(-32768))

    def body_hi(i, base):
        cand = base + (jnp.int32(1) << (14 - i))
        cnt = _sum16((khi >= cand.astype(jnp.int16)).astype(jnp.int16))
        return jnp.where(cnt >= TOP_K, cand, base)

    hi = jax.lax.fori_loop(0, 15, body_hi, base0)
    hi16 = hi.astype(jnp.int16)

    cgt = _sum16((khi > hi16).astype(jnp.int16))
    rem = float(TOP_K) - cgt
    lo16 = (jnp.bitwise_and(key, 0xFFFF) - 32768).astype(jnp.int16)
    kl = jnp.where(khi == hi16, lo16, jnp.int16(-32768))

    def body_lo(i, base):
        cand = base + (jnp.int32(1) << (15 - i))
        cnt = _sum16((kl >= cand.astype(jnp.int16)).astype(jnp.int16))
        return jnp.where(cnt >= rem, cand, base)

    lo = jax.lax.fori_loop(0, 16, body_lo, jnp.full_like(hi, -32768))

    kth_key = (hi << 16) + (lo + 32768)

    p = jnp.where(key >= kth_key, jnp.exp(s - rowmax), 0.0)
    denom = jnp.sum(p, axis=1, keepdims=True)
    msg = jax.lax.dot_general(
        p.astype(jnp.bfloat16), stb_ref[0], (((1,), (0,)), ((), ())),
        preferred_element_type=jnp.float32) / denom
    m_ref[0] = msg
    mb_ref[0] = msg.astype(jnp.bfloat16)


def _coalition_kernel(pnb_ref, pnf_ref, msgb_ref, msgblk_ref, o_ref):
    sim = jax.lax.dot_general(pnb_ref[0], pnf_ref[0], (((1,), (1,)), ((), ())),
                              preferred_element_type=jnp.float32)
    hit = sim > THRESH
    cnt = jnp.sum(hit.astype(jnp.float32), axis=1, keepdims=True)
    comb = jax.lax.dot_general(
        hit.astype(jnp.bfloat16), msgb_ref[0], (((1,), (0,)), ((), ())),
        preferred_element_type=jnp.float32) / (cnt + 1e-8)
    o_ref[0] = 0.8 * msgblk_ref[0] + 0.2 * comb


@jax.jit
def kernel(agent_states, agent_actions, Wq, bq, Wk, bk, Wc, bc):
    nb = N // RB
    q, k, pn, stb = pl.pallas_call(
        _proj_kernel,
        grid=(B,),
        in_specs=[
            pl.BlockSpec((1, N, D), lambda b: (b, 0, 0)),
            pl.BlockSpec((1, N, D), lambda b: (b, 0, 0)),
            pl.BlockSpec((D, PD), lambda b: (0, 0)),
            pl.BlockSpec((PD,), lambda b: (0,)),
            pl.BlockSpec((D, PD), lambda b: (0, 0)),
            pl.BlockSpec((PD,), lambda b: (0,)),
            pl.BlockSpec((D, PD), lambda b: (0, 0)),
            pl.BlockSpec((PD,), lambda b: (0,)),
        ],
        out_specs=[
            pl.BlockSpec((1, N, PD), lambda b: (b, 0, 0)),
            pl.BlockSpec((1, N, PD), lambda b: (b, 0, 0)),
            pl.BlockSpec((1, N, PD), lambda b: (b, 0, 0)),
            pl.BlockSpec((1, N, D), lambda b: (b, 0, 0)),
        ],
        out_shape=[
            jax.ShapeDtypeStruct((B, N, PD), jnp.float32),
            jax.ShapeDtypeStruct((B, N, PD), jnp.float32),
            jax.ShapeDtypeStruct((B, N, PD), jnp.bfloat16),
            jax.ShapeDtypeStruct((B, N, D), jnp.bfloat16),
        ],
    )(agent_actions, agent_states, Wq, bq, Wk, bk, Wc, bc)

    messages, messages_b = pl.pallas_call(
        _messages_kernel,
        grid=(B, nb),
        in_specs=[
            pl.BlockSpec((1, RB, PD), lambda b, i: (b, i, 0)),
            pl.BlockSpec((1, N, PD), lambda b, i: (b, 0, 0)),
            pl.BlockSpec((1, N, D), lambda b, i: (b, 0, 0)),
        ],
        out_specs=[
            pl.BlockSpec((1, RB, D), lambda b, i: (b, i, 0)),
            pl.BlockSpec((1, RB, D), lambda b, i: (b, i, 0)),
        ],
        out_shape=[
            jax.ShapeDtypeStruct((B, N, D), jnp.float32),
            jax.ShapeDtypeStruct((B, N, D), jnp.bfloat16),
        ],
    )(q, k, stb)

    out = pl.pallas_call(
        _coalition_kernel,
        grid=(B, nb),
        in_specs=[
            pl.BlockSpec((1, RB, PD), lambda b, i: (b, i, 0)),
            pl.BlockSpec((1, N, PD), lambda b, i: (b, 0, 0)),
            pl.BlockSpec((1, N, D), lambda b, i: (b, 0, 0)),
            pl.BlockSpec((1, RB, D), lambda b, i: (b, i, 0)),
        ],
        out_specs=pl.BlockSpec((1, RB, D), lambda b, i: (b, i, 0)),
        out_shape=jax.ShapeDtypeStruct((B, N, D), jnp.float32),
    )(pn, pn, messages_b, messages)

    return out
